# baseline (speedup 1.0000x reference)
import functools

import jax
import jax.numpy as jnp
from jax import lax
from jax.experimental import pallas as pl
from jax.experimental.pallas import tpu as pltpu
from jax.experimental.pallas import tpu_sc as plsc

D = 32
SEQ = 50
BATCH = 4096
NW = 32
B_PER_W = BATCH // NW
BC = 16
NCHUNK = B_PER_W // BC
DRAIN_COLS = 128
CHUNK_BYTES = BC * SEQ * 128
N_DRAIN = CHUNK_BYTES // (SEQ * DRAIN_COLS * 4)

_mesh = plsc.VectorSubcoreMesh(core_axis_name="c", subcore_axis_name="s")


@functools.partial(
    pl.kernel,
    mesh=_mesh,
    out_type=(
        jax.ShapeDtypeStruct((BATCH, SEQ, D), jnp.float32),
        jax.ShapeDtypeStruct((BC * SEQ, 32), jnp.float32),
    ),
    scratch_types=[
        pltpu.VMEM((SEQ, B_PER_W), jnp.int32),
        pltpu.VMEM((BC * SEQ, 32), jnp.float32),
        pltpu.SemaphoreType.DMA,
        pltpu.SemaphoreType.DMA,
    ],
)
def _gather_kernel(idx_hbm, table_hbm, out_hbm, dummy_hbm, idx_v, buf_v, sem, osem):
    wid = lax.axis_index("s") * 2 + lax.axis_index("c")
    bcol = pl.multiple_of(wid * B_PER_W, B_PER_W)
    pltpu.sync_copy(idx_hbm.at[:, pl.ds(bcol, B_PER_W)], idx_v)

    def drain(which_sem):
        pltpu.make_async_copy(dummy_hbm, buf_v, which_sem).wait()

    def chunk_body(c, _):
        def s_body(s, _):
            vec = idx_v[s, pl.ds(c * BC, 16)]
            for j in range(16):
                r = vec[j]
                t = lax.shift_right_logical(r, 3)
                sub = lax.bitwise_and(r, 7)
                pltpu.async_copy(
                    table_hbm.at[t, sub],
                    buf_v.at[j * SEQ + s],
                    sem,
                )
            return 0

        lax.fori_loop(0, SEQ, s_body, 0)
        drain(sem)
        for j in range(BC):
            pltpu.async_copy(
                buf_v.at[pl.ds(j * SEQ, SEQ)],
                out_hbm.at[bcol + c * BC + j],
                osem,
            )
        drain(osem)
        return 0

    lax.fori_loop(0, NCHUNK, chunk_body, 0)


def kernel(input, embeddings):
    idx_t = input.T.astype(jnp.int32)
    table3 = embeddings.reshape(125000, 8, 32)
    out, _ = _gather_kernel(idx_t, table3)
    return out


# speedup vs baseline: 1.1711x; 1.1711x over previous
import functools

import jax
import jax.numpy as jnp
from jax import lax
from jax.experimental import pallas as pl
from jax.experimental.pallas import tpu as pltpu
from jax.experimental.pallas import tpu_sc as plsc

D = 32
SEQ = 50
SEQ_P = 56
BATCH = 4096
NW = 32
B_PER_W = BATCH // NW
HC = 8
NPAIR = B_PER_W // (2 * HC)
BUF_ROWS = HC * SEQ_P

_mesh = plsc.VectorSubcoreMesh(core_axis_name="c", subcore_axis_name="s")


@functools.partial(
    pl.kernel,
    mesh=_mesh,
    out_type=(
        jax.ShapeDtypeStruct((BATCH * SEQ_P, 128), jnp.float32),
        jax.ShapeDtypeStruct((SEQ, 128), jnp.int32),
    ),
    scratch_types=[
        pltpu.VMEM((SEQ, B_PER_W), jnp.int32),
        pltpu.VMEM((BUF_ROWS, 128), jnp.float32),
        pltpu.VMEM((BUF_ROWS, 128), jnp.float32),
        pltpu.SemaphoreType.DMA,
        pltpu.SemaphoreType.DMA,
        pltpu.SemaphoreType.DMA,
        pltpu.SemaphoreType.DMA,
    ],
)
def _gather_kernel(
    idx_hbm, table_hbm, out_hbm, dummy_hbm,
    idx_v, buf_a, buf_b, sem_a, sem_b, osem_a, osem_b,
):
    wid = lax.axis_index("s") * 2 + lax.axis_index("c")
    bcol = pl.multiple_of(wid * B_PER_W, B_PER_W)
    pltpu.sync_copy(idx_hbm.at[:, pl.ds(bcol, B_PER_W)], idx_v)

    def enqueue_pair(p):
        def s_body(s, _):
            vec = idx_v[s, pl.ds(p * 16, 16)]
            for j in range(16):
                r = vec[j]
                t = lax.shift_right_logical(r, 3)
                sub = lax.bitwise_and(r, 7)
                buf, sem = (buf_a, sem_a) if j < HC else (buf_b, sem_b)
                pltpu.async_copy(
                    table_hbm.at[t, sub],
                    buf.at[(j % HC) * SEQ_P + s, pl.ds(0, 32)],
                    sem,
                )
            return 0

        lax.fori_loop(0, SEQ, s_body, 0)

    def drain_gather(sem):
        for _ in range(2):
            pltpu.make_async_copy(dummy_hbm, idx_v, sem).wait()

    def flush(buf, b0, osem):
        row0 = pl.multiple_of(b0 * SEQ_P, 8)
        pltpu.async_copy(buf, out_hbm.at[pl.ds(row0, BUF_ROWS)], osem)

    def drain_flush(osem):
        pltpu.make_async_copy(out_hbm.at[pl.ds(0, BUF_ROWS)], buf_a, osem).wait()

    for p in range(NPAIR):
        if p > 0:
            drain_flush(osem_a)
            drain_flush(osem_b)
        enqueue_pair(p)
        drain_gather(sem_a)
        flush(buf_a, bcol + p * 16, osem_a)
        drain_gather(sem_b)
        flush(buf_b, bcol + p * 16 + HC, osem_b)
    drain_flush(osem_a)
    drain_flush(osem_b)


def kernel(input, embeddings):
    idx_t = input.T.astype(jnp.int32)
    table3 = embeddings.reshape(125000, 8, 32)
    out2d, _ = _gather_kernel(idx_t, table3)
    out4 = out2d.reshape(BATCH, SEQ_P, 128)
    return out4[:, :SEQ, :D]
